# Initial kernel scaffold; baseline (speedup 1.0000x reference)
import jax
import jax.numpy as jnp
from jax.experimental import pallas as pl


def kernel(u, i, user_emb, user_W1, user_b1, user_W2, user_b2, item_emb, item_W1, item_b1, item_W2, item_b2):
    raise NotImplementedError("write your pallas kernel here")


# speedup vs baseline: 6.5259x; 6.5259x over previous
import functools

import jax
import jax.numpy as jnp
from jax import lax
from jax.experimental import pallas as pl
from jax.experimental.pallas import tpu as pltpu
from jax.experimental.pallas import tpu_sc as plsc

_B = 16384
_D = 128
_HID = 64
_OUT = 32
_NC = 2
_NS = 16
_NW = _NC * _NS
_BPW = _B // _NW
_CH = 128
_NCH = _BPW // _CH


def _gather_body(uemb, iemb, uidx, iidx, urows, irows,
                 idx_u, idx_i, buf0, buf1, sem0, sem1):
    cid = lax.axis_index("c")
    sid = lax.axis_index("s")
    wid = sid * _NC + cid
    base = wid * _BPW
    pltpu.sync_copy(uidx.at[pl.ds(wid * _NCH, _NCH)], idx_u)
    pltpu.sync_copy(iidx.at[pl.ds(wid * _NCH, _NCH)], idx_i)
    bufs = (buf0, buf1)
    sems = (sem0, sem1)
    cps = [None, None]
    for c in range(_NCH):
        cps[c % 2] = pltpu.async_copy(uemb.at[idx_u.at[c]], bufs[c % 2], sems[c % 2])
        if c > 0:
            cps[(c - 1) % 2].wait()
            pltpu.sync_copy(bufs[(c - 1) % 2], urows.at[pl.ds(base + (c - 1) * _CH, _CH)])
    cps[(_NCH - 1) % 2].wait()
    pltpu.sync_copy(bufs[(_NCH - 1) % 2], urows.at[pl.ds(base + (_NCH - 1) * _CH, _CH)])
    for c in range(_NCH):
        cps[c % 2] = pltpu.async_copy(iemb.at[idx_i.at[c]], bufs[c % 2], sems[c % 2])
        if c > 0:
            cps[(c - 1) % 2].wait()
            pltpu.sync_copy(bufs[(c - 1) % 2], irows.at[pl.ds(base + (c - 1) * _CH, _CH)])
    cps[(_NCH - 1) % 2].wait()
    pltpu.sync_copy(bufs[(_NCH - 1) % 2], irows.at[pl.ds(base + (_NCH - 1) * _CH, _CH)])


def _sc_gather(uemb, iemb, uidx, iidx):
    mesh = plsc.VectorSubcoreMesh(core_axis_name="c", subcore_axis_name="s",
                                  num_cores=_NC, num_subcores=_NS)
    fn = pl.kernel(
        _gather_body,
        out_type=[jax.ShapeDtypeStruct((_B, _D), jnp.float32),
                  jax.ShapeDtypeStruct((_B, _D), jnp.float32)],
        mesh=mesh,
        scratch_types=[
            pltpu.VMEM((_NCH, _CH), jnp.int32),
            pltpu.VMEM((_NCH, _CH), jnp.int32),
            pltpu.VMEM((_CH, _D), jnp.float32),
            pltpu.VMEM((_CH, _D), jnp.float32),
            pltpu.SemaphoreType.DMA,
            pltpu.SemaphoreType.DMA,
        ],
    )
    return fn(uemb, iemb, uidx, iidx)


_BB = 2048


def _mlp_body(ur, ir, uw1, ub1, uw2, ub2, iw1, ib1, iw2, ib2, out):
    ux = jnp.maximum(jnp.dot(ur[...], uw1[...], preferred_element_type=jnp.float32)
                     + ub1[...], 0.0)
    ue = jnp.dot(ux, uw2[...], preferred_element_type=jnp.float32) + ub2[...]
    ix = jnp.maximum(jnp.dot(ir[...], iw1[...], preferred_element_type=jnp.float32)
                     + ib1[...], 0.0)
    ie = jnp.dot(ix, iw2[...], preferred_element_type=jnp.float32) + ib2[...]
    out[...] = jnp.sum(ue * ie, axis=1)


def _tc_mlp(urows, irows, uw1t, ub1, uw2t, ub2, iw1t, ib1, iw2t, ib2):
    grid = (_B // _BB,)
    full = lambda shape: pl.BlockSpec(shape, lambda b: (0,) * len(shape))
    return pl.pallas_call(
        _mlp_body,
        grid=grid,
        in_specs=[
            pl.BlockSpec((_BB, _D), lambda b: (b, 0)),
            pl.BlockSpec((_BB, _D), lambda b: (b, 0)),
            full((_D, _HID)), full((1, _HID)), full((_HID, _OUT)), full((1, _OUT)),
            full((_D, _HID)), full((1, _HID)), full((_HID, _OUT)), full((1, _OUT)),
        ],
        out_specs=pl.BlockSpec((_BB,), lambda b: (b,)),
        out_shape=jax.ShapeDtypeStruct((_B,), jnp.float32),
    )(urows, irows, uw1t, ub1, uw2t, ub2, iw1t, ib1, iw2t, ib2)


def kernel(u, i, user_emb, user_W1, user_b1, user_W2, user_b2,
           item_emb, item_W1, item_b1, item_W2, item_b2):
    uidx = u.astype(jnp.int32).reshape(_NW * _NCH, _CH)
    iidx = i.astype(jnp.int32).reshape(_NW * _NCH, _CH)
    urows, irows = _sc_gather(user_emb, item_emb, uidx, iidx)
    return _tc_mlp(
        urows, irows,
        user_W1.T, user_b1.reshape(1, _HID), user_W2.T, user_b2.reshape(1, _OUT),
        item_W1.T, item_b1.reshape(1, _HID), item_W2.T, item_b2.reshape(1, _OUT),
    )
